# baseline (speedup 1.0000x reference)
import functools

import jax
import jax.numpy as jnp
from jax import lax
from jax.experimental import pallas as pl
from jax.experimental.pallas import tpu as pltpu
from jax.experimental.pallas import tpu_sc as plsc

N = 10000
D = 128
E = 320000

NUM_SC = 2
NUM_TILES = 16
NW = NUM_SC * NUM_TILES
EDGES_PER_TILE = E // NW
CHUNK = 128
NCH = 80
E_PAD = NW * NCH * CHUNK
CROWS = E_PAD // CHUNK
PAD_ROWS = 8
ACC_ROWS = N + PAD_ROWS
IO_TILES = 10
IO_ROWS = N // IO_TILES
WB_SIZES = (128, 128, 128, 128, 128, 128, 128, 104)
HALF = NCH // 2
DEG_W = 16

ROW_BLOCK = 1000
GRID = N // ROW_BLOCK


def _make_seg():
    mesh = plsc.VectorSubcoreMesh(core_axis_name="c", subcore_axis_name="s")
    out_type = jax.ShapeDtypeStruct((NUM_SC, N, D), jnp.float32)
    scratch = [
        pltpu.VMEM((HALF, CHUNK), jnp.int32),
        pltpu.VMEM((HALF, CHUNK), jnp.int32),
        pltpu.VMEM((CHUNK, D), jnp.float32),
        pltpu.VMEM((CHUNK, D), jnp.float32),
        pltpu.VMEM_SHARED((ACC_ROWS, D), jnp.float32),
        pltpu.SemaphoreType.DMA,
        pltpu.SemaphoreType.DMA,
        pltpu.SemaphoreType.DMA,
        pltpu.SemaphoreType.DMA,
    ]

    def body(y_hbm, src_hbm, dst_hbm, zacc_hbm, psum_hbm,
             src_v, dst_v, stage0, stage1, acc_sh,
             gsem0, gsem1, ssem0, ssem1):
        cid = lax.axis_index("c")
        sid = lax.axis_index("s")
        wid = cid * NUM_TILES + sid
        cbase = wid * NCH
        rbase = sid * IO_ROWS
        stages = (stage0, stage1)
        gsems = (gsem0, gsem1)
        ssems = (ssem0, ssem1)

        @pl.when(sid < IO_TILES)
        def _init():
            pltpu.sync_copy(zacc_hbm.at[pl.ds(0, CHUNK)], stage0)
            r = rbase
            for w in WB_SIZES:
                pltpu.sync_copy(stage0.at[pl.ds(0, w)],
                                acc_sh.at[pl.ds(r, w)])
                r += w

        @pl.when(sid == IO_TILES)
        def _init_pad():
            pltpu.sync_copy(zacc_hbm.at[pl.ds(0, CHUNK)], stage0)
            pltpu.sync_copy(stage0.at[pl.ds(0, PAD_ROWS)],
                            acc_sh.at[pl.ds(N, PAD_ROWS)])
        plsc.subcore_barrier()

        def gather_start(k, b):
            pltpu.async_copy(y_hbm.at[src_v.at[k]], stages[b], gsems[b])

        def gather_wait(b):
            pltpu.make_async_copy(y_hbm.at[pl.ds(0, CHUNK)], stages[b],
                                  gsems[b]).wait()

        def scat_start(k, b):
            pltpu.async_copy(stages[b], acc_sh.at[dst_v.at[k]], ssems[b],
                             add=True)

        def scat_wait(b):
            pltpu.make_async_copy(stages[b], acc_sh.at[dst_v.at[0]],
                                  ssems[b]).wait()

        for h in range(2):
            pltpu.sync_copy(src_hbm.at[pl.ds(cbase + h * HALF, HALF)], src_v)
            pltpu.sync_copy(dst_hbm.at[pl.ds(cbase + h * HALF, HALF)], dst_v)

            gather_start(0, 0)
            gather_wait(0)
            scat_start(0, 0)
            gather_start(1, 1)

            def pair(i, carry):
                k0 = 2 * i + 1
                gather_wait(1)
                scat_start(k0, 1)
                scat_wait(0)
                gather_start(k0 + 1, 0)
                k1 = 2 * i + 2
                gather_wait(0)
                scat_start(k1, 0)
                scat_wait(1)
                gather_start(k1 + 1, 1)
                return carry
            lax.fori_loop(0, (HALF - 2) // 2, pair, 0)

            gather_wait(1)
            scat_start(HALF - 1, 1)
            scat_wait(0)
            scat_wait(1)

        plsc.subcore_barrier()

        @pl.when(sid < IO_TILES)
        def _writeback():
            r = rbase
            for w in WB_SIZES:
                pltpu.sync_copy(acc_sh.at[pl.ds(r, w)],
                                stage0.at[pl.ds(0, w)])
                pltpu.sync_copy(stage0.at[pl.ds(0, w)],
                                psum_hbm.at[cid, pl.ds(r, w)])
                r += w

    return pl.kernel(body, out_type=out_type, mesh=mesh, scratch_types=scratch)


def _make_deg3():
    mesh = plsc.VectorSubcoreMesh(core_axis_name="c", subcore_axis_name="s")
    out_type = [jax.ShapeDtypeStruct((NUM_SC, N, D), jnp.float32)] * 3
    scratch = [
        pltpu.VMEM((NCH, CHUNK), jnp.int32),
        pltpu.VMEM((CHUNK, D), jnp.float32),
        pltpu.VMEM_SHARED((ACC_ROWS, D), jnp.float32),
        pltpu.SemaphoreType.DMA,
        pltpu.SemaphoreType.DMA,
        pltpu.SemaphoreType.DMA,
        pltpu.SemaphoreType.DMA,
    ]
    NSEM = 4

    def body(dst0_hbm, dst1_hbm, dst2_hbm, zacc_hbm, ones_hbm,
             degp0_hbm, degp1_hbm, degp2_hbm,
             dst_v, ones_v, deg_sh, *ssems):
        degps = (degp0_hbm, degp1_hbm, degp2_hbm)
        cid = lax.axis_index("c")
        sid = lax.axis_index("s")
        wid = cid * NUM_TILES + sid
        cbase = wid * NCH
        rbase = sid * IO_ROWS

        def scat_start(k, b):
            pltpu.async_copy(ones_v, deg_sh.at[dst_v.at[k]], ssems[b],
                             add=True)

        def scat_wait(b):
            pltpu.make_async_copy(ones_v, deg_sh.at[dst_v.at[0]],
                                  ssems[b]).wait()

        for r, dst_hbm in enumerate((dst0_hbm, dst1_hbm, dst2_hbm)):
            @pl.when(sid < IO_TILES)
            def _init():
                pltpu.sync_copy(zacc_hbm.at[pl.ds(0, CHUNK)], ones_v)
                rr = rbase
                for w in WB_SIZES:
                    pltpu.sync_copy(ones_v.at[pl.ds(0, w)],
                                    deg_sh.at[pl.ds(rr, w)])
                    rr += w

            @pl.when(sid == IO_TILES)
            def _init_pad():
                pltpu.sync_copy(zacc_hbm.at[pl.ds(0, CHUNK)], ones_v)
                pltpu.sync_copy(ones_v.at[pl.ds(0, PAD_ROWS)],
                                deg_sh.at[pl.ds(N, PAD_ROWS)])

            pltpu.sync_copy(ones_hbm, ones_v)
            pltpu.sync_copy(dst_hbm.at[pl.ds(cbase, NCH)], dst_v)
            plsc.subcore_barrier()

            for b in range(NSEM):
                scat_start(b, b)

            def quad(i, carry):
                for b in range(NSEM):
                    scat_wait(b)
                    scat_start(NSEM * i + NSEM + b, b)
                return carry
            lax.fori_loop(0, (NCH - NSEM) // NSEM, quad, 0)
            for b in range(NSEM):
                scat_wait(b)

            plsc.subcore_barrier()

            @pl.when(sid < IO_TILES)
            def _writeback():
                rr = rbase
                for w in WB_SIZES:
                    pltpu.sync_copy(deg_sh.at[pl.ds(rr, w)],
                                    ones_v.at[pl.ds(0, w)])
                    pltpu.sync_copy(ones_v.at[pl.ds(0, w)],
                                    degps[r].at[cid, pl.ds(rr, w)])
                    rr += w
            plsc.subcore_barrier()

    return pl.kernel(body, out_type=out_type, mesh=mesh, scratch_types=scratch)


_seg = _make_seg()
_deg3 = _make_deg3()


def _mm1_body(x_ref, wn_ref, ws0_ref, ws1_ref, ws2_ref, b_ref,
              y0_ref, y1_ref, y2_ref, s_ref):
    x = x_ref[...]
    yn = jnp.dot(x, wn_ref[...], preferred_element_type=jnp.float32)
    y0_ref[...] = yn[:, 0:D]
    y1_ref[...] = yn[:, D:2 * D]
    y2_ref[...] = yn[:, 2 * D:3 * D]
    ws = ws0_ref[...] + ws1_ref[...] + ws2_ref[...]
    s_ref[...] = jnp.dot(x, ws, preferred_element_type=jnp.float32) + b_ref[...]


def _mean(p_ref, d_ref):
    p = p_ref[...]
    d = d_ref[...]
    deg = d[0, :, 0:1] + d[1, :, 0:1]
    return (p[0] + p[1]) / jnp.maximum(deg, 1.0)


def _mid_body(s1_ref, p0_ref, p1_ref, p2_ref, d0_ref, d1_ref, d2_ref,
              wn_ref, ws0_ref, ws1_ref, ws2_ref, b_ref,
              y0_ref, y1_ref, y2_ref, s_ref):
    h = (s1_ref[...] + _mean(p0_ref, d0_ref) + _mean(p1_ref, d1_ref)
         + _mean(p2_ref, d2_ref))
    h = jnp.maximum(h, 0.0)
    yn = jnp.dot(h, wn_ref[...], preferred_element_type=jnp.float32)
    y0_ref[...] = yn[:, 0:D]
    y1_ref[...] = yn[:, D:2 * D]
    y2_ref[...] = yn[:, 2 * D:3 * D]
    ws = ws0_ref[...] + ws1_ref[...] + ws2_ref[...]
    s_ref[...] = jnp.dot(h, ws, preferred_element_type=jnp.float32) + b_ref[...]


def _out_body(s2_ref, p0_ref, p1_ref, p2_ref, d0_ref, d1_ref, d2_ref, o_ref):
    o_ref[...] = (s2_ref[...] + _mean(p0_ref, d0_ref) + _mean(p1_ref, d1_ref)
                  + _mean(p2_ref, d2_ref))


_ROW = pl.BlockSpec((ROW_BLOCK, D), lambda i: (i, 0))
_ROW3 = pl.BlockSpec((ROW_BLOCK, 3 * D), lambda i: (i, 0))
_PSUM = pl.BlockSpec((NUM_SC, ROW_BLOCK, D), lambda i: (0, i, 0))
_DEGP = pl.BlockSpec((NUM_SC, ROW_BLOCK, D), lambda i: (0, i, 0))
_WN = pl.BlockSpec((D, 3 * D), lambda i: (0, 0))
_WS = pl.BlockSpec((D, D), lambda i: (0, 0))
_B = pl.BlockSpec((1, D), lambda i: (0, 0))

_ROWOUT = [jax.ShapeDtypeStruct((N, D), jnp.float32)] * 4

_mm1 = pl.pallas_call(
    _mm1_body, grid=(GRID,),
    in_specs=[_ROW, _WN, _WS, _WS, _WS, _B],
    out_specs=[_ROW, _ROW, _ROW, _ROW],
    out_shape=_ROWOUT,
)

_mid = pl.pallas_call(
    _mid_body, grid=(GRID,),
    in_specs=[_ROW, _PSUM, _PSUM, _PSUM, _DEGP, _DEGP, _DEGP,
              _WN, _WS, _WS, _WS, _B],
    out_specs=[_ROW, _ROW, _ROW, _ROW],
    out_shape=_ROWOUT,
)

_outc = pl.pallas_call(
    _out_body, grid=(GRID,),
    in_specs=[_ROW, _PSUM, _PSUM, _PSUM, _DEGP, _DEGP, _DEGP],
    out_specs=_ROW,
    out_shape=jax.ShapeDtypeStruct((N, D), jnp.float32),
)


def kernel(x, edge_index_r0, edge_index_r1, edge_index_r2,
           W1_self_r0, W1_neigh_r0, b1_r0,
           W1_self_r1, W1_neigh_r1, b1_r1,
           W1_self_r2, W1_neigh_r2, b1_r2,
           W2_self_r0, W2_neigh_r0, b2_r0,
           W2_self_r1, W2_neigh_r1, b2_r1,
           W2_self_r2, W2_neigh_r2, b2_r2):
    wn1 = jnp.concatenate([W1_neigh_r0, W1_neigh_r1, W1_neigh_r2], axis=1)
    wn2 = jnp.concatenate([W2_neigh_r0, W2_neigh_r1, W2_neigh_r2], axis=1)
    b1 = (b1_r0 + b1_r1 + b1_r2).reshape(1, D)
    b2 = (b2_r0 + b2_r1 + b2_r2).reshape(1, D)
    zacc = jnp.zeros((N, D), jnp.float32)
    ones = jnp.ones((CHUNK, D), jnp.float32)

    def prep(ei):
        pad_src = jnp.arange(E_PAD - E, dtype=jnp.int32) % N
        src = jnp.concatenate([ei[0], pad_src]).reshape(CROWS, CHUNK)
        dst = jnp.concatenate(
            [ei[1], jnp.full((E_PAD - E,), N, jnp.int32)]).reshape(CROWS, CHUNK)
        return src, dst

    src0, dst0 = prep(edge_index_r0)
    src1, dst1 = prep(edge_index_r1)
    src2, dst2 = prep(edge_index_r2)

    y10, y11, y12, s1 = _mm1(x, wn1, W1_self_r0, W1_self_r1, W1_self_r2, b1)
    d0, d1, d2 = _deg3(dst0, dst1, dst2, zacc, ones)
    p0 = _seg(y10, src0, dst0, zacc)
    p1 = _seg(y11, src1, dst1, zacc)
    p2 = _seg(y12, src2, dst2, zacc)
    y20, y21, y22, s2 = _mid(s1, p0, p1, p2, d0, d1, d2,
                             wn2, W2_self_r0, W2_self_r1, W2_self_r2, b2)
    q0 = _seg(y20, src0, dst0, zacc)
    q1 = _seg(y21, src1, dst1, zacc)
    q2 = _seg(y22, src2, dst2, zacc)
    return _outc(s2, q0, q1, q2, d0, d1, d2)


# speedup vs baseline: 1.0043x; 1.0043x over previous
import functools

import jax
import jax.numpy as jnp
from jax import lax
from jax.experimental import pallas as pl
from jax.experimental.pallas import tpu as pltpu
from jax.experimental.pallas import tpu_sc as plsc

N = 10000
D = 128
E = 320000

NUM_SC = 2
NUM_TILES = 16
NW = NUM_SC * NUM_TILES
CHUNK = 128
NCH = 80
E_PAD = NW * NCH * CHUNK
CROWS = E_PAD // CHUNK
PAD_ROWS = 8
ACC_ROWS = N + PAD_ROWS
IO_TILES = 10
IO_ROWS = N // IO_TILES
WB_SIZES = (128, 128, 128, 128, 128, 128, 128, 104)
HALF = NCH // 2

ROW_BLOCK = 1000
GRID = N // ROW_BLOCK


def _make_seg():
    mesh = plsc.VectorSubcoreMesh(core_axis_name="c", subcore_axis_name="s")
    out_type = jax.ShapeDtypeStruct((NUM_SC, N, D), jnp.float32)
    scratch = [
        pltpu.VMEM((HALF, CHUNK), jnp.int32),
        pltpu.VMEM((HALF, CHUNK), jnp.int32),
        pltpu.VMEM((CHUNK, D), jnp.float32),
        pltpu.VMEM((CHUNK, D), jnp.float32),
        pltpu.VMEM_SHARED((ACC_ROWS, D), jnp.float32),
        pltpu.SemaphoreType.DMA,
        pltpu.SemaphoreType.DMA,
        pltpu.SemaphoreType.DMA,
        pltpu.SemaphoreType.DMA,
    ]

    def body(y_hbm, src_hbm, dst_hbm, zacc_hbm, psum_hbm,
             src_v, dst_v, stage0, stage1, acc_sh,
             gsem0, gsem1, ssem0, ssem1):
        cid = lax.axis_index("c")
        sid = lax.axis_index("s")
        wid = cid * NUM_TILES + sid
        cbase = wid * NCH
        rbase = sid * IO_ROWS
        stages = (stage0, stage1)
        gsems = (gsem0, gsem1)
        ssems = (ssem0, ssem1)

        @pl.when(sid < IO_TILES)
        def _init():
            pltpu.sync_copy(zacc_hbm.at[pl.ds(0, CHUNK)], stage0)
            r = rbase
            for w in WB_SIZES:
                pltpu.sync_copy(stage0.at[pl.ds(0, w)],
                                acc_sh.at[pl.ds(r, w)])
                r += w

        @pl.when(sid == IO_TILES)
        def _init_pad():
            pltpu.sync_copy(zacc_hbm.at[pl.ds(0, CHUNK)], stage0)
            pltpu.sync_copy(stage0.at[pl.ds(0, PAD_ROWS)],
                            acc_sh.at[pl.ds(N, PAD_ROWS)])
        plsc.subcore_barrier()

        def gather_start(k, b):
            pltpu.async_copy(y_hbm.at[src_v.at[k]], stages[b], gsems[b])

        def gather_wait(b):
            pltpu.make_async_copy(y_hbm.at[pl.ds(0, CHUNK)], stages[b],
                                  gsems[b]).wait()

        def scat_start(k, b):
            pltpu.async_copy(stages[b], acc_sh.at[dst_v.at[k]], ssems[b],
                             add=True)

        def scat_wait(b):
            pltpu.make_async_copy(stages[b], acc_sh.at[dst_v.at[0]],
                                  ssems[b]).wait()

        for h in range(2):
            pltpu.sync_copy(src_hbm.at[pl.ds(cbase + h * HALF, HALF)], src_v)
            pltpu.sync_copy(dst_hbm.at[pl.ds(cbase + h * HALF, HALF)], dst_v)

            gather_start(0, 0)
            gather_wait(0)
            scat_start(0, 0)
            gather_start(1, 1)

            def pair(i, carry):
                k0 = 2 * i + 1
                gather_wait(1)
                scat_start(k0, 1)
                scat_wait(0)
                gather_start(k0 + 1, 0)
                k1 = 2 * i + 2
                gather_wait(0)
                scat_start(k1, 0)
                scat_wait(1)
                gather_start(k1 + 1, 1)
                return carry
            lax.fori_loop(0, (HALF - 2) // 2, pair, 0)

            gather_wait(1)
            scat_start(HALF - 1, 1)
            scat_wait(0)
            scat_wait(1)

        plsc.subcore_barrier()

        @pl.when(sid < IO_TILES)
        def _writeback():
            r = rbase
            for w in WB_SIZES:
                pltpu.sync_copy(acc_sh.at[pl.ds(r, w)],
                                stage0.at[pl.ds(0, w)])
                pltpu.sync_copy(stage0.at[pl.ds(0, w)],
                                psum_hbm.at[cid, pl.ds(r, w)])
                r += w

    return pl.kernel(body, out_type=out_type, mesh=mesh, scratch_types=scratch)


def _make_deg3():
    mesh = plsc.VectorSubcoreMesh(core_axis_name="c", subcore_axis_name="s")
    out_type = [jax.ShapeDtypeStruct((NUM_SC, N, D), jnp.float32)] * 3
    scratch = [
        pltpu.VMEM((NCH, CHUNK), jnp.int32),
        pltpu.VMEM((CHUNK, D), jnp.float32),
        pltpu.VMEM_SHARED((ACC_ROWS, D), jnp.float32),
        pltpu.SemaphoreType.DMA,
        pltpu.SemaphoreType.DMA,
        pltpu.SemaphoreType.DMA,
        pltpu.SemaphoreType.DMA,
    ]
    NSEM = 4

    def body(dst0_hbm, dst1_hbm, dst2_hbm, zacc_hbm, ones_hbm,
             degp0_hbm, degp1_hbm, degp2_hbm,
             dst_v, ones_v, deg_sh, *ssems):
        degps = (degp0_hbm, degp1_hbm, degp2_hbm)
        cid = lax.axis_index("c")
        sid = lax.axis_index("s")
        wid = cid * NUM_TILES + sid
        cbase = wid * NCH
        rbase = sid * IO_ROWS

        def scat_start(k, b):
            pltpu.async_copy(ones_v, deg_sh.at[dst_v.at[k]], ssems[b],
                             add=True)

        def scat_wait(b):
            pltpu.make_async_copy(ones_v, deg_sh.at[dst_v.at[0]],
                                  ssems[b]).wait()

        for r, dst_hbm in enumerate((dst0_hbm, dst1_hbm, dst2_hbm)):
            @pl.when(sid < IO_TILES)
            def _init():
                pltpu.sync_copy(zacc_hbm.at[pl.ds(0, CHUNK)], ones_v)
                rr = rbase
                for w in WB_SIZES:
                    pltpu.sync_copy(ones_v.at[pl.ds(0, w)],
                                    deg_sh.at[pl.ds(rr, w)])
                    rr += w

            @pl.when(sid == IO_TILES)
            def _init_pad():
                pltpu.sync_copy(zacc_hbm.at[pl.ds(0, CHUNK)], ones_v)
                pltpu.sync_copy(ones_v.at[pl.ds(0, PAD_ROWS)],
                                deg_sh.at[pl.ds(N, PAD_ROWS)])

            pltpu.sync_copy(ones_hbm, ones_v)
            pltpu.sync_copy(dst_hbm.at[pl.ds(cbase, NCH)], dst_v)
            plsc.subcore_barrier()

            for b in range(NSEM):
                scat_start(b, b)

            def quad(i, carry):
                for b in range(NSEM):
                    scat_wait(b)
                    scat_start(NSEM * i + NSEM + b, b)
                return carry
            lax.fori_loop(0, (NCH - NSEM) // NSEM, quad, 0)
            for b in range(NSEM):
                scat_wait(b)

            plsc.subcore_barrier()

            @pl.when(sid < IO_TILES)
            def _writeback():
                rr = rbase
                for w in WB_SIZES:
                    pltpu.sync_copy(deg_sh.at[pl.ds(rr, w)],
                                    ones_v.at[pl.ds(0, w)])
                    pltpu.sync_copy(ones_v.at[pl.ds(0, w)],
                                    degps[r].at[cid, pl.ds(rr, w)])
                    rr += w
            plsc.subcore_barrier()

    return pl.kernel(body, out_type=out_type, mesh=mesh, scratch_types=scratch)


_seg = _make_seg()
_deg3 = _make_deg3()


def _mm1_body(x_ref, wn_ref, ws0_ref, ws1_ref, ws2_ref, b_ref,
              y0_ref, y1_ref, y2_ref, s_ref):
    x = x_ref[...]
    yn = jnp.dot(x, wn_ref[...], preferred_element_type=jnp.float32)
    y0_ref[...] = yn[:, 0:D]
    y1_ref[...] = yn[:, D:2 * D]
    y2_ref[...] = yn[:, 2 * D:3 * D]
    ws = ws0_ref[...] + ws1_ref[...] + ws2_ref[...]
    s_ref[...] = jnp.dot(x, ws, preferred_element_type=jnp.float32) + b_ref[...]


def _mean(p_ref, d_ref):
    p = p_ref[...]
    d = d_ref[...]
    deg = d[0, :, 0:1] + d[1, :, 0:1]
    return (p[0] + p[1]) / jnp.maximum(deg, 1.0)


def _mid_body(s1_ref, p0_ref, p1_ref, p2_ref, d0_ref, d1_ref, d2_ref,
              wn_ref, ws0_ref, ws1_ref, ws2_ref, b_ref,
              y0_ref, y1_ref, y2_ref, s_ref):
    h = (s1_ref[...] + _mean(p0_ref, d0_ref) + _mean(p1_ref, d1_ref)
         + _mean(p2_ref, d2_ref))
    h = jnp.maximum(h, 0.0)
    yn = jnp.dot(h, wn_ref[...], preferred_element_type=jnp.float32)
    y0_ref[...] = yn[:, 0:D]
    y1_ref[...] = yn[:, D:2 * D]
    y2_ref[...] = yn[:, 2 * D:3 * D]
    ws = ws0_ref[...] + ws1_ref[...] + ws2_ref[...]
    s_ref[...] = jnp.dot(h, ws, preferred_element_type=jnp.float32) + b_ref[...]


def _out_body(s2_ref, p0_ref, p1_ref, p2_ref, d0_ref, d1_ref, d2_ref, o_ref):
    o_ref[...] = (s2_ref[...] + _mean(p0_ref, d0_ref) + _mean(p1_ref, d1_ref)
                  + _mean(p2_ref, d2_ref))


_ROW = pl.BlockSpec((ROW_BLOCK, D), lambda i: (i, 0))
_ROW3 = pl.BlockSpec((ROW_BLOCK, 3 * D), lambda i: (i, 0))
_PSUM = pl.BlockSpec((NUM_SC, ROW_BLOCK, D), lambda i: (0, i, 0))
_DEGP = pl.BlockSpec((NUM_SC, ROW_BLOCK, D), lambda i: (0, i, 0))
_WN = pl.BlockSpec((D, 3 * D), lambda i: (0, 0))
_WS = pl.BlockSpec((D, D), lambda i: (0, 0))
_B = pl.BlockSpec((1, D), lambda i: (0, 0))

_ROWOUT = [jax.ShapeDtypeStruct((N, D), jnp.float32)] * 4

_mm1 = pl.pallas_call(
    _mm1_body, grid=(GRID,),
    in_specs=[_ROW, _WN, _WS, _WS, _WS, _B],
    out_specs=[_ROW, _ROW, _ROW, _ROW],
    out_shape=_ROWOUT,
)

_mid = pl.pallas_call(
    _mid_body, grid=(GRID,),
    in_specs=[_ROW, _PSUM, _PSUM, _PSUM, _DEGP, _DEGP, _DEGP,
              _WN, _WS, _WS, _WS, _B],
    out_specs=[_ROW, _ROW, _ROW, _ROW],
    out_shape=_ROWOUT,
)

_outc = pl.pallas_call(
    _out_body, grid=(GRID,),
    in_specs=[_ROW, _PSUM, _PSUM, _PSUM, _DEGP, _DEGP, _DEGP],
    out_specs=_ROW,
    out_shape=jax.ShapeDtypeStruct((N, D), jnp.float32),
)


def kernel(x, edge_index_r0, edge_index_r1, edge_index_r2,
           W1_self_r0, W1_neigh_r0, b1_r0,
           W1_self_r1, W1_neigh_r1, b1_r1,
           W1_self_r2, W1_neigh_r2, b1_r2,
           W2_self_r0, W2_neigh_r0, b2_r0,
           W2_self_r1, W2_neigh_r1, b2_r1,
           W2_self_r2, W2_neigh_r2, b2_r2):
    wn1 = jnp.concatenate([W1_neigh_r0, W1_neigh_r1, W1_neigh_r2], axis=1)
    wn2 = jnp.concatenate([W2_neigh_r0, W2_neigh_r1, W2_neigh_r2], axis=1)
    b1 = (b1_r0 + b1_r1 + b1_r2).reshape(1, D)
    b2 = (b2_r0 + b2_r1 + b2_r2).reshape(1, D)
    zacc = jnp.zeros((N, D), jnp.float32)
    ones = jnp.ones((CHUNK, D), jnp.float32)

    def prep(ei):
        pad_src = jnp.arange(E_PAD - E, dtype=jnp.int32) % N
        src = jnp.concatenate([ei[0], pad_src]).reshape(CROWS, CHUNK)
        dst = jnp.concatenate(
            [ei[1], jnp.full((E_PAD - E,), N, jnp.int32)]).reshape(CROWS, CHUNK)
        return src, dst

    src0, dst0 = prep(edge_index_r0)
    src1, dst1 = prep(edge_index_r1)
    src2, dst2 = prep(edge_index_r2)

    y10, y11, y12, s1 = _mm1(x, wn1, W1_self_r0, W1_self_r1, W1_self_r2, b1)
    d0, d1, d2 = _deg3(dst0, dst1, dst2, zacc, ones)
    p0 = _seg(y10, src0, dst0, zacc)
    p1 = _seg(y11, src1, dst1, zacc)
    p2 = _seg(y12, src2, dst2, zacc)
    y20, y21, y22, s2 = _mid(s1, p0, p1, p2, d0, d1, d2,
                             wn2, W2_self_r0, W2_self_r1, W2_self_r2, b2)
    q0 = _seg(y20, src0, dst0, zacc)
    q1 = _seg(y21, src1, dst1, zacc)
    q2 = _seg(y22, src2, dst2, zacc)
    return _outc(s2, q0, q1, q2, d0, d1, d2)
